# Initial kernel scaffold; baseline (speedup 1.0000x reference)
import jax
import jax.numpy as jnp
from jax.experimental import pallas as pl


def kernel(x, kohonen_weights, G_fwd, G_rev):
    raise NotImplementedError("write your pallas kernel here")


# speedup vs baseline: 1.3372x; 1.3372x over previous
import functools

import jax
import jax.numpy as jnp
from jax import lax
from jax.experimental import pallas as pl
from jax.experimental.pallas import tpu as pltpu
from jax.experimental.pallas import tpu_sc as plsc


def _argmin_body(x_ref, w_ref, x2_ref, w2_ref, out_ref, best_val, best_idx):
    h = pl.program_id(1)
    nh = pl.num_programs(1)
    ht = w_ref.shape[0]

    s = lax.dot_general(
        x_ref[...],
        w_ref[...],
        dimension_numbers=(((1,), (1,)), ((), ())),
        preferred_element_type=jnp.float32,
    )
    d2 = (x2_ref[...] + w2_ref[...]) - 2.0 * s
    d2 = jnp.maximum(d2, 0.0)

    tmin = jnp.min(d2, axis=1, keepdims=True)
    iota = lax.broadcasted_iota(jnp.int32, d2.shape, 1)
    larg = jnp.min(jnp.where(d2 == tmin, iota, ht), axis=1, keepdims=True)
    gidx = larg + h * ht

    @pl.when(h == 0)
    def _():
        best_val[...] = tmin
        best_idx[...] = gidx

    @pl.when(h > 0)
    def _():
        upd = tmin < best_val[...]
        best_idx[...] = jnp.where(upd, gidx, best_idx[...])
        best_val[...] = jnp.where(upd, tmin, best_val[...])

    @pl.when(h == nh - 1)
    def _():
        out_ref[...] = best_idx[...]


def _tc_winners(x, w, x2, w2, bt=1024, ht=1024):
    b, d = x.shape
    hh = w.shape[0]
    grid = (b // bt, hh // ht)
    return pl.pallas_call(
        _argmin_body,
        grid=grid,
        in_specs=[
            pl.BlockSpec((bt, d), lambda i, j: (i, 0)),
            pl.BlockSpec((ht, d), lambda i, j: (j, 0)),
            pl.BlockSpec((bt, 1), lambda i, j: (i, 0)),
            pl.BlockSpec((1, ht), lambda i, j: (0, j)),
        ],
        out_specs=pl.BlockSpec((bt, 1), lambda i, j: (i, 0)),
        out_shape=jax.ShapeDtypeStruct((b, 1), jnp.int32),
        scratch_shapes=[
            pltpu.VMEM((bt, 1), jnp.float32),
            pltpu.VMEM((bt, 1), jnp.int32),
        ],
    )(x, w, x2, w2)


_NC, _NS = 2, 16
_NW = _NC * _NS


def _sc_gather_pair(tab_f, tab_r, idx):
    b = idx.shape[0]
    df = tab_f.shape[1]
    dr = tab_r.shape[1]
    b_per_w = b // _NW
    cf = 32
    n_chunks = b_per_w // cf
    mesh = plsc.VectorSubcoreMesh(core_axis_name="c", subcore_axis_name="s")

    @functools.partial(
        pl.kernel,
        mesh=mesh,
        out_type=[
            jax.ShapeDtypeStruct((b, df), jnp.float32),
            jax.ShapeDtypeStruct((b, dr), jnp.float32),
        ],
        scratch_types=[
            pltpu.VMEM((b_per_w,), jnp.int32),
            pltpu.VMEM((cf, df), jnp.float32),
            pltpu.VMEM((b_per_w, dr), jnp.float32),
            pltpu.SemaphoreType.DMA,
        ],
    )
    def k(tf_hbm, tr_hbm, idx_hbm, of_hbm, or_hbm, idx_v, rf_v, rr_v, sem):
        wid = lax.axis_index("s") * _NC + lax.axis_index("c")
        base = wid * b_per_w
        pltpu.sync_copy(idx_hbm.at[pl.ds(base, b_per_w)], idx_v)
        pltpu.async_copy(tr_hbm.at[idx_v], rr_v, sem).wait()
        pltpu.sync_copy(rr_v, or_hbm.at[pl.ds(base, b_per_w)])
        for c in range(n_chunks):
            pltpu.async_copy(
                tf_hbm.at[idx_v.at[pl.ds(c * cf, cf)]], rf_v, sem
            ).wait()
            pltpu.sync_copy(rf_v, of_hbm.at[pl.ds(base + c * cf, cf)])

    return k(tab_f, tab_r, idx)


def kernel(x, kohonen_weights, G_fwd, G_rev):
    x = x.reshape(x.shape[0], -1)
    b = x.shape[0]
    o = G_fwd.shape[0]

    x2 = jnp.sum(x * x, axis=1, keepdims=True)
    w2 = jnp.sum(kohonen_weights * kohonen_weights, axis=1)[None, :]

    winners2d = _tc_winners(x, kohonen_weights, x2, w2)
    winners = winners2d.reshape(b)

    o_pad = ((o + 63) // 64) * 64
    tab_f = jnp.pad(G_fwd.T, ((0, 0), (0, o_pad - o)))
    tab_r = G_rev.T

    out_f, recos = _sc_gather_pair(tab_f, tab_r, winners)
    output = out_f[:, :o]
    return (output, recos, winners)
